# baseline (speedup 1.0000x reference)
import dataclasses

import jax
import jax.numpy as jnp
from jax import lax
from jax.experimental import pallas as pl
from jax.experimental.pallas import tpu as pltpu
from jax.experimental.pallas import tpu_sc as plsc

K = 5
B = 128
N = 100000
RT = 96
RB = 16
NRB = RT // RB
SC_ROWS = B - RT
L = 16


def _tc_body(y_vec_ref, x_ref, out_ref):
    j = pl.program_id(0)
    yv = y_vec_ref[...]
    x = x_ref[...]
    io = jax.lax.broadcasted_iota(jnp.int32, (RB, N), 1)

    eqy = io == yv
    v = jnp.max(jnp.where(eqy, x, -jnp.inf), axis=1, keepdims=True)
    hits = jnp.logical_or(
        x > v, jnp.logical_and(x == v, io < yv)
    ).astype(jnp.float32)
    cnt = jnp.sum(hits, axis=1, keepdims=True)
    part = jnp.sum((cnt < float(K)).astype(jnp.int32))

    @pl.when(j == 0)
    def _first():
        out_ref[0, 0] = part

    @pl.when(j > 0)
    def _rest():
        out_ref[0, 0] += part


def _tc_count(y_vec, y_pred):
    out = pl.pallas_call(
        _tc_body,
        grid=(NRB,),
        in_specs=[
            pl.BlockSpec((RB, 1), lambda j: (j, 0)),
            pl.BlockSpec((RB, N), lambda j: (j, 0)),
        ],
        out_specs=pl.BlockSpec(memory_space=pltpu.MemorySpace.SMEM),
        out_shape=jax.ShapeDtypeStruct((1, 1), jnp.int32),
        compiler_params=pltpu.CompilerParams(
            dimension_semantics=("arbitrary",),
        ),
    )(y_vec, y_pred)
    return out.reshape(())


def _sc_body(ypred_hbm, y_hbm, out_hbm, y_v, row_v, acc_v, flag_v, sem):
    wid = lax.axis_index("s") * 2 + lax.axis_index("c")
    i = RT + wid
    il = lax.iota(jnp.int32, L)

    pltpu.sync_copy(y_hbm.at[pl.ds((i // L) * L, L)], y_v)
    yi = jnp.sum(jnp.where(il == i % L, y_v[...], 0))

    pltpu.async_copy(ypred_hbm.at[i], row_v, sem).wait()

    voff = (yi // L) * L
    vchunk = row_v[pl.ds(voff, L)]
    v = jnp.sum(jnp.where(il == yi - voff, vchunk, 0.0))

    acc_v[...] = jnp.zeros((L,), jnp.float32)

    @pl.loop(0, N, step=L)
    def _(o):
        c = row_v[pl.ds(o, L)]
        lt = il < yi - o
        m = jnp.logical_or(c > v, jnp.logical_and(c == v, lt))
        acc_v[...] += jnp.where(m, 1.0, 0.0)

    cnt = jnp.sum(acc_v[...])
    flag = jnp.where(cnt < float(K), 1, 0)
    flag_v[...] = jnp.broadcast_to(flag, (L,)).astype(jnp.int32)
    pltpu.sync_copy(flag_v, out_hbm.at[wid])


_sc_count_cache = []


def _sc_count(y_pred, y32):
    if not _sc_count_cache:
        mesh = plsc.VectorSubcoreMesh(
            core_axis_name="c", subcore_axis_name="s", num_cores=2, num_subcores=16
        )
        cp = pltpu.CompilerParams()
        if "needs_layout_passes" in pltpu.CompilerParams.__dataclass_fields__:
            cp = dataclasses.replace(cp, needs_layout_passes=False)
        _sc_count_cache.append(
            pl.kernel(
                _sc_body,
                out_type=jax.ShapeDtypeStruct((SC_ROWS, L), jnp.int32),
                mesh=mesh,
                compiler_params=cp,
                scratch_types=[
                    pltpu.VMEM((L,), jnp.int32),
                    pltpu.VMEM((N,), jnp.float32),
                    pltpu.VMEM((L,), jnp.float32),
                    pltpu.VMEM((L,), jnp.int32),
                    pltpu.SemaphoreType.DMA,
                ],
            )
        )
    return _sc_count_cache[0](y_pred, y32)


def kernel(y_pred, y):
    y32 = y.astype(jnp.int32)
    sc_flags = _sc_count(y_pred, y32)
    tc_part = _tc_count(y32.reshape(B, 1), y_pred)
    return tc_part + jnp.sum(sc_flags[:, 0])


# speedup vs baseline: 1.6304x; 1.6304x over previous
import jax
import jax.numpy as jnp
from jax.experimental import pallas as pl
from jax.experimental.pallas import tpu as pltpu

K = 5
B = 128
N = 100000
RB = 32
NRB = B // RB
MAX_OFF = (N - 128) // 128 * 128
TAIL0 = (N // 128) * 128


def _issue(y_smem, ypred_hbm, gbuf, sem, j, parity):
    def _start(r, _):
        gr = j * RB + r
        off = jnp.minimum((y_smem[gr] // 128) * 128, MAX_OFF)
        pltpu.make_async_copy(
            ypred_hbm.at[pl.ds((gr // 8) * 8, 8), pl.ds(off, 128)],
            gbuf.at[parity, r],
            sem.at[parity],
        ).start()
        return 0

    jax.lax.fori_loop(0, RB, _start, 0)


def _drain(ypred_hbm, gbuf, sem, parity):
    def _wait(r, _):
        pltpu.make_async_copy(
            ypred_hbm.at[pl.ds(0, 8), pl.ds(0, 128)],
            gbuf.at[parity, 0],
            sem.at[parity],
        ).wait()
        return 0

    jax.lax.fori_loop(0, RB, _wait, 0)


def _tc_body(y_smem, y_vec_ref, x_ref, ypred_hbm, out_ref, gbuf, sem):
    j = pl.program_id(0)
    parity = jax.lax.rem(j, 2)

    @pl.when(j == 0)
    def _prime():
        _issue(y_smem, ypred_hbm, gbuf, sem, 0, 0)

    _drain(ypred_hbm, gbuf, sem, parity)

    @pl.when(j < NRB - 1)
    def _ahead():
        _issue(y_smem, ypred_hbm, gbuf, sem, j + 1, 1 - parity)

    yv = y_vec_ref[...]
    x = x_ref[...]

    off_vec = jnp.minimum((yv // 128) * 128, MAX_OFF)
    lane = (yv - off_vec).reshape(RB, 1, 1)
    ri = jax.lax.rem(jax.lax.broadcasted_iota(jnp.int32, (RB, 8, 128), 0), 8)
    si = jax.lax.broadcasted_iota(jnp.int32, (RB, 8, 128), 1)
    li = jax.lax.broadcasted_iota(jnp.int32, (RB, 8, 128), 2)
    sel = jnp.logical_and(si == ri, li == lane)
    g = jnp.where(parity == 0, gbuf[0], gbuf[1])
    v_dma = jnp.sum(
        jnp.sum(jnp.where(sel, g, 0.0), axis=2), axis=1, keepdims=True
    )
    xt = x_ref[:, N - 128:]
    iot = (N - 128) + jax.lax.broadcasted_iota(jnp.int32, (RB, 128), 1)
    v_tail = jnp.max(
        jnp.where(iot == yv, xt, -jnp.inf), axis=1, keepdims=True
    )
    v = jnp.where(yv >= TAIL0, v_tail, v_dma)

    io = jax.lax.broadcasted_iota(jnp.int32, (RB, N), 1)
    hits = jnp.logical_or(
        x > v, jnp.logical_and(x == v, io < yv)
    ).astype(jnp.float32)
    cnt = jnp.sum(hits, axis=1, keepdims=True)
    part = jnp.sum((cnt < float(K)).astype(jnp.int32))

    @pl.when(j == 0)
    def _first():
        out_ref[0, 0] = part

    @pl.when(j > 0)
    def _rest():
        out_ref[0, 0] += part


def kernel(y_pred, y):
    y32 = y.astype(jnp.int32)
    grid_spec = pltpu.PrefetchScalarGridSpec(
        num_scalar_prefetch=1,
        grid=(NRB,),
        in_specs=[
            pl.BlockSpec((RB, 1), lambda j, y_s: (j, 0)),
            pl.BlockSpec((RB, N), lambda j, y_s: (j, 0)),
            pl.BlockSpec(memory_space=pltpu.MemorySpace.HBM),
        ],
        out_specs=pl.BlockSpec(memory_space=pltpu.MemorySpace.SMEM),
        scratch_shapes=[
            pltpu.VMEM((2, RB, 8, 128), jnp.float32),
            pltpu.SemaphoreType.DMA((2,)),
        ],
    )
    out = pl.pallas_call(
        _tc_body,
        grid_spec=grid_spec,
        out_shape=jax.ShapeDtypeStruct((1, 1), jnp.int32),
        compiler_params=pltpu.CompilerParams(
            dimension_semantics=("arbitrary",),
        ),
    )(y32, y32.reshape(B, 1), y_pred, y_pred)
    return out.reshape(())
